# baseline (speedup 1.0000x reference)
import functools

import jax
import jax.numpy as jnp
from jax import lax
from jax.experimental import pallas as pl
from jax.experimental.pallas import tpu as pltpu
from jax.experimental.pallas import tpu_sc as plsc

N = 10000
E = 320000
D = 128

NC = 2
NS = 16
NW = NC * NS
EPW = E // NW
K = 80
CH = EPW // K
RPT = 624
TAIL = N - NS * RPT
ZR = 24


def _scatter_body(h_hbm, pk_hbm, out_hbm,
                  pkb, sb, db, rows, zbuf, acc, semI, semG, semS):
    c = lax.axis_index("c")
    s = lax.axis_index("s")
    wid = c * NS + s

    for r in range(ZR):
        for q in range(D // 16):
            zbuf[r, pl.ds(q * 16, 16)] = jnp.zeros((16,), jnp.float32)
    row0 = s * RPT
    for t in range(RPT // ZR):
        pltpu.sync_copy(zbuf, acc.at[pl.ds(row0 + t * ZR, ZR)])

    @pl.when(s == 0)
    def _zero_tail():
        pltpu.sync_copy(zbuf.at[pl.ds(0, TAIL)], acc.at[pl.ds(NS * RPT, TAIL)])

    plsc.subcore_barrier()

    def pkload(k, b):
        pltpu.async_copy(pk_hbm.at[pl.ds((wid * CH + k) * K, K)], pkb[b],
                         semI[b])

    def iwait(k, b):
        pltpu.make_async_copy(pk_hbm.at[pl.ds((wid * CH + k) * K, K)],
                              pkb[b], semI[b]).wait()

    def unpack(b):
        for q in range(K // 16):
            v = pkb[b][pl.ds(q * 16, 16)]
            sb[b][pl.ds(q * 16, 16)] = v >> 14
            db[b][pl.ds(q * 16, 16)] = v & 16383

    def gather(b):
        pltpu.async_copy(h_hbm.at[sb[b]], rows[b], semG[b])

    def gwait(b):
        pltpu.make_async_copy(h_hbm.at[sb[b]], rows[b], semG[b]).wait()

    def scat(b):
        pltpu.async_copy(rows[b], acc.at[db[b]], semS[b], add=True)

    def swait(b):
        pltpu.make_async_copy(rows[b], acc.at[db[b]], semS[b]).wait()

    for b in range(3):
        pkload(b, b)
    for b in range(3):
        iwait(b, b)
        unpack(b)
        gather(b)
        pkload(b + 3, b)

    def round_(j, carry):
        base = 3 * j
        for b in range(3):
            gwait(b)
            scat(b)
        for b in range(3):
            swait(b)
            iwait(base + 3 + b, b)
            unpack(b)
            gather(b)

            @pl.when(base + 6 + b < CH)
            def _pf():
                pkload(base + 6 + b, b)

        return carry

    nrounds = (CH - 5) // 3
    lax.fori_loop(0, nrounds, round_, 0)

    e = 3 * nrounds
    for b in range(3):
        gwait(b)
        scat(b)
    for b in range(2):
        swait(b)
        iwait(e + 3 + b, b)
        unpack(b)
        gather(b)
    for b in range(2):
        gwait(b)
        scat(b)
    for b in range(3):
        swait(b)
    plsc.subcore_barrier()

    pltpu.sync_copy(acc.at[pl.ds(row0, RPT)],
                    out_hbm.at[c, pl.ds(row0, RPT)])

    @pl.when(s == 0)
    def _write_tail():
        pltpu.sync_copy(acc.at[pl.ds(NS * RPT, TAIL)],
                        out_hbm.at[c, pl.ds(NS * RPT, TAIL)])


@jax.jit
def _scatter_partials(h, packed):
    mesh = plsc.VectorSubcoreMesh(core_axis_name="c", subcore_axis_name="s")
    f = pl.kernel(
        _scatter_body,
        out_type=jax.ShapeDtypeStruct((NC, N, D), jnp.float32),
        mesh=mesh,
        scratch_types=[
            [pltpu.VMEM((K,), jnp.int32) for _ in range(3)],
            [pltpu.VMEM((K,), jnp.int32) for _ in range(3)],
            [pltpu.VMEM((K,), jnp.int32) for _ in range(3)],
            [pltpu.VMEM((K, D), jnp.float32) for _ in range(3)],
            pltpu.VMEM((ZR, D), jnp.float32),
            pltpu.VMEM_SHARED((N, D), jnp.float32),
            [pltpu.SemaphoreType.DMA for _ in range(3)],
            [pltpu.SemaphoreType.DMA for _ in range(3)],
            [pltpu.SemaphoreType.DMA for _ in range(3)],
        ],
    )
    return f(h, packed)


BN = 2000


def _mm_body(x_ref, w_ref, b_ref, o_ref, *, relu):
    acc = lax.dot_general(x_ref[...], w_ref[...],
                          dimension_numbers=(((1,), (1,)), ((), ())),
                          preferred_element_type=jnp.float32,
                          precision=lax.Precision.HIGHEST)
    acc = acc + b_ref[...]
    o_ref[...] = jnp.maximum(acc, 0.0) if relu else acc


def _mm_agg_body(x_ref, p0_ref, p1_ref, w_ref, b_ref, o_ref, *, relu):
    hh = x_ref[...] + p0_ref[...] + p1_ref[...]
    acc = lax.dot_general(hh, w_ref[...],
                          dimension_numbers=(((1,), (1,)), ((), ())),
                          preferred_element_type=jnp.float32,
                          precision=lax.Precision.HIGHEST)
    acc = acc + b_ref[...]
    o_ref[...] = jnp.maximum(acc, 0.0) if relu else acc


_row_spec = pl.BlockSpec((BN, D), lambda i: (i, 0))
_full_spec = pl.BlockSpec((D, D), lambda i: (0, 0))
_b_spec = pl.BlockSpec((1, D), lambda i: (0, 0))


def _linear(x, w, b, relu=False):
    return pl.pallas_call(
        functools.partial(_mm_body, relu=relu),
        grid=(N // BN,),
        in_specs=[_row_spec, _full_spec, _b_spec],
        out_specs=_row_spec,
        out_shape=jax.ShapeDtypeStruct((N, D), jnp.float32),
    )(x, w, b.reshape(1, D))


def _linear_agg(x, p0, p1, w, b, relu=False):
    return pl.pallas_call(
        functools.partial(_mm_agg_body, relu=relu),
        grid=(N // BN,),
        in_specs=[_row_spec, _row_spec, _row_spec, _full_spec, _b_spec],
        out_specs=_row_spec,
        out_shape=jax.ShapeDtypeStruct((N, D), jnp.float32),
    )(x, p0, p1, w, b.reshape(1, D))


def kernel(x, edge_index, W_pre, b_pre, W1, b1, W2, b2):
    packed = ((edge_index[0] << 14) | edge_index[1]).reshape(E)
    h0 = _linear(x, W_pre, b_pre)
    p = _scatter_partials(h0, packed)
    h1 = _linear_agg(h0, p[0], p[1], W1, b1, relu=True)
    q = _scatter_partials(h1, packed)
    return _linear_agg(h1, q[0], q[1], W2, b2, relu=False)


# speedup vs baseline: 1.0541x; 1.0541x over previous
import functools

import jax
import jax.numpy as jnp
from jax import lax
from jax.experimental import pallas as pl
from jax.experimental.pallas import tpu as pltpu
from jax.experimental.pallas import tpu_sc as plsc

N = 10000
E = 320000
D = 128

NC = 2
NS = 16
NW = NC * NS
EPW = E // NW
K = 80
CH = EPW // K
RPT = 624
TAIL = N - NS * RPT
ZR = 24


def _scatter_body(h_hbm, pk_hbm, out_hbm,
                  pk, sb0, db0, sb1, db1, rows0, rows1, zbuf, acc,
                  sem0, sem1):
    c = lax.axis_index("c")
    s = lax.axis_index("s")
    wid = c * NS + s

    pltpu.sync_copy(pk_hbm.at[wid], pk)

    for r in range(ZR):
        for q in range(D // 16):
            zbuf[r, pl.ds(q * 16, 16)] = jnp.zeros((16,), jnp.float32)
    row0 = s * RPT
    for t in range(RPT // ZR):
        pltpu.sync_copy(zbuf, acc.at[pl.ds(row0 + t * ZR, ZR)])

    @pl.when(s == 0)
    def _zero_tail():
        pltpu.sync_copy(zbuf.at[pl.ds(0, TAIL)], acc.at[pl.ds(NS * RPT, TAIL)])

    plsc.subcore_barrier()

    def unpack(k, sb, db):
        for q in range(K // 16):
            v = pk[k, pl.ds(q * 16, 16)]
            sb[pl.ds(q * 16, 16)] = v >> 14
            db[pl.ds(q * 16, 16)] = v & 16383

    def gather(sb, buf, sem):
        pltpu.async_copy(h_hbm.at[sb], buf, sem)

    def gwait(sb, buf, sem):
        pltpu.make_async_copy(h_hbm.at[sb], buf, sem).wait()

    def scat(buf, db):
        pltpu.sync_copy(buf, acc.at[db], add=True)

    unpack(0, sb0, db0)
    gather(sb0, rows0, sem0)

    def pipe(j, carry):
        k0 = 2 * j
        unpack(k0 + 1, sb1, db1)
        gather(sb1, rows1, sem1)
        gwait(sb0, rows0, sem0)
        scat(rows0, db0)
        unpack(k0 + 2, sb0, db0)
        gather(sb0, rows0, sem0)
        gwait(sb1, rows1, sem1)
        scat(rows1, db1)
        return carry

    lax.fori_loop(0, (CH - 1) // 2, pipe, 0)
    gwait(sb0, rows0, sem0)
    scat(rows0, db0)
    plsc.subcore_barrier()

    pltpu.sync_copy(acc.at[pl.ds(row0, RPT)],
                    out_hbm.at[c, pl.ds(row0, RPT)])

    @pl.when(s == 0)
    def _write_tail():
        pltpu.sync_copy(acc.at[pl.ds(NS * RPT, TAIL)],
                        out_hbm.at[c, pl.ds(NS * RPT, TAIL)])


@jax.jit
def _scatter_partials(h, packed):
    mesh = plsc.VectorSubcoreMesh(core_axis_name="c", subcore_axis_name="s")
    f = pl.kernel(
        _scatter_body,
        out_type=jax.ShapeDtypeStruct((NC, N, D), jnp.float32),
        mesh=mesh,
        scratch_types=[
            pltpu.VMEM((CH, K), jnp.int32),
            pltpu.VMEM((K,), jnp.int32),
            pltpu.VMEM((K,), jnp.int32),
            pltpu.VMEM((K,), jnp.int32),
            pltpu.VMEM((K,), jnp.int32),
            pltpu.VMEM((K, D), jnp.float32),
            pltpu.VMEM((K, D), jnp.float32),
            pltpu.VMEM((ZR, D), jnp.float32),
            pltpu.VMEM_SHARED((N, D), jnp.float32),
            pltpu.SemaphoreType.DMA,
            pltpu.SemaphoreType.DMA,
        ],
    )
    return f(h, packed)


BN = 2000


def _mm_body(x_ref, w_ref, b_ref, o_ref, *, relu):
    acc = lax.dot_general(x_ref[...], w_ref[...],
                          dimension_numbers=(((1,), (1,)), ((), ())),
                          preferred_element_type=jnp.float32,
                          precision=lax.Precision.HIGHEST)
    acc = acc + b_ref[...]
    o_ref[...] = jnp.maximum(acc, 0.0) if relu else acc


def _mm_agg_body(x_ref, p0_ref, p1_ref, w_ref, b_ref, o_ref, *, relu):
    hh = x_ref[...] + p0_ref[0] + p1_ref[0]
    acc = lax.dot_general(hh, w_ref[...],
                          dimension_numbers=(((1,), (1,)), ((), ())),
                          preferred_element_type=jnp.float32,
                          precision=lax.Precision.HIGHEST)
    acc = acc + b_ref[...]
    o_ref[...] = jnp.maximum(acc, 0.0) if relu else acc


_row_spec = pl.BlockSpec((BN, D), lambda i: (i, 0))
_p0_spec = pl.BlockSpec((1, BN, D), lambda i: (0, i, 0))
_p1_spec = pl.BlockSpec((1, BN, D), lambda i: (1, i, 0))
_full_spec = pl.BlockSpec((D, D), lambda i: (0, 0))
_b_spec = pl.BlockSpec((1, D), lambda i: (0, 0))


def _linear(x, w, b, relu=False):
    return pl.pallas_call(
        functools.partial(_mm_body, relu=relu),
        grid=(N // BN,),
        in_specs=[_row_spec, _full_spec, _b_spec],
        out_specs=_row_spec,
        out_shape=jax.ShapeDtypeStruct((N, D), jnp.float32),
    )(x, w, b.reshape(1, D))


def _linear_agg(x, p, w, b, relu=False):
    return pl.pallas_call(
        functools.partial(_mm_agg_body, relu=relu),
        grid=(N // BN,),
        in_specs=[_row_spec, _p0_spec, _p1_spec, _full_spec, _b_spec],
        out_specs=_row_spec,
        out_shape=jax.ShapeDtypeStruct((N, D), jnp.float32),
    )(x, p, p, w, b.reshape(1, D))


def kernel(x, edge_index, W_pre, b_pre, W1, b1, W2, b2):
    packed = ((edge_index[0] << 14) | edge_index[1]).reshape(NW, CH, K)
    h0 = _linear(x, W_pre, b_pre)
    p = _scatter_partials(h0, packed)
    h1 = _linear_agg(h0, p, W1, b1, relu=True)
    q = _scatter_partials(h1, packed)
    return _linear_agg(h1, q, W2, b2, relu=False)


# speedup vs baseline: 1.0867x; 1.0309x over previous
import functools

import jax
import jax.numpy as jnp
from jax import lax
from jax.experimental import pallas as pl
from jax.experimental.pallas import tpu as pltpu
from jax.experimental.pallas import tpu_sc as plsc

N = 10000
E = 320000
D = 128

NC = 2
NS = 16
NW = NC * NS
EPW = E // NW
K = 80
CH = EPW // K
RPT = 624
TAIL = N - NS * RPT
ZR = 24


def _scatter_body(h_hbm, pk_hbm, out_hbm,
                  pk, sb0, db0, sb1, db1, rows0, rows1, zbuf, acc,
                  sem0, sem1, semP, semZ):
    c = lax.axis_index("c")
    s = lax.axis_index("s")
    wid = c * NS + s

    pltpu.async_copy(pk_hbm.at[wid], pk, semP)

    for r in range(ZR):
        for q in range(D // 16):
            zbuf[r, pl.ds(q * 16, 16)] = jnp.zeros((16,), jnp.float32)
    row0 = s * RPT
    for t in range(RPT // ZR):
        pltpu.async_copy(zbuf, acc.at[pl.ds(row0 + t * ZR, ZR)], semZ)

    @pl.when(s == 0)
    def _zero_tail():
        pltpu.async_copy(zbuf.at[pl.ds(0, TAIL)], acc.at[pl.ds(NS * RPT, TAIL)],
                         semZ)

    pltpu.make_async_copy(pk_hbm.at[wid], pk, semP).wait()

    def unpack(k, sb, db):
        for q in range(K // 16):
            v = pk[k, pl.ds(q * 16, 16)]
            sb[pl.ds(q * 16, 16)] = v >> 14
            db[pl.ds(q * 16, 16)] = v & 16383

    def gather(sb, buf, sem):
        pltpu.async_copy(h_hbm.at[sb], buf, sem)

    def gwait(sb, buf, sem):
        pltpu.make_async_copy(h_hbm.at[sb], buf, sem).wait()

    def scat(buf, db):
        pltpu.sync_copy(buf, acc.at[db], add=True)

    unpack(0, sb0, db0)
    gather(sb0, rows0, sem0)

    for t in range(RPT // ZR):
        pltpu.make_async_copy(zbuf, acc.at[pl.ds(row0 + t * ZR, ZR)],
                              semZ).wait()

    @pl.when(s == 0)
    def _zero_tail_wait():
        pltpu.make_async_copy(zbuf.at[pl.ds(0, TAIL)],
                              acc.at[pl.ds(NS * RPT, TAIL)], semZ).wait()

    plsc.subcore_barrier()

    def pipe(j, carry):
        k0 = 2 * j
        unpack(k0 + 1, sb1, db1)
        gather(sb1, rows1, sem1)
        gwait(sb0, rows0, sem0)
        scat(rows0, db0)
        unpack(k0 + 2, sb0, db0)
        gather(sb0, rows0, sem0)
        gwait(sb1, rows1, sem1)
        scat(rows1, db1)
        return carry

    lax.fori_loop(0, (CH - 1) // 2, pipe, 0)
    gwait(sb0, rows0, sem0)
    scat(rows0, db0)
    plsc.subcore_barrier()

    pltpu.sync_copy(acc.at[pl.ds(row0, RPT)],
                    out_hbm.at[c, pl.ds(row0, RPT)])

    @pl.when(s == 0)
    def _write_tail():
        pltpu.sync_copy(acc.at[pl.ds(NS * RPT, TAIL)],
                        out_hbm.at[c, pl.ds(NS * RPT, TAIL)])


@jax.jit
def _scatter_partials(h, packed):
    mesh = plsc.VectorSubcoreMesh(core_axis_name="c", subcore_axis_name="s")
    f = pl.kernel(
        _scatter_body,
        out_type=jax.ShapeDtypeStruct((NC, N, D), jnp.float32),
        mesh=mesh,
        scratch_types=[
            pltpu.VMEM((CH, K), jnp.int32),
            pltpu.VMEM((K,), jnp.int32),
            pltpu.VMEM((K,), jnp.int32),
            pltpu.VMEM((K,), jnp.int32),
            pltpu.VMEM((K,), jnp.int32),
            pltpu.VMEM((K, D), jnp.float32),
            pltpu.VMEM((K, D), jnp.float32),
            pltpu.VMEM((ZR, D), jnp.float32),
            pltpu.VMEM_SHARED((N, D), jnp.float32),
            pltpu.SemaphoreType.DMA,
            pltpu.SemaphoreType.DMA,
            pltpu.SemaphoreType.DMA,
            pltpu.SemaphoreType.DMA,
        ],
    )
    return f(h, packed)


BN = 2000


def _mm_body(x_ref, w_ref, b_ref, o_ref, *, relu):
    acc = lax.dot_general(x_ref[...], w_ref[...],
                          dimension_numbers=(((1,), (1,)), ((), ())),
                          preferred_element_type=jnp.float32,
                          precision=lax.Precision.HIGHEST)
    acc = acc + b_ref[...]
    o_ref[...] = jnp.maximum(acc, 0.0) if relu else acc


def _mm_agg_body(x_ref, p0_ref, p1_ref, w_ref, b_ref, o_ref, *, relu):
    hh = x_ref[...] + p0_ref[0] + p1_ref[0]
    acc = lax.dot_general(hh, w_ref[...],
                          dimension_numbers=(((1,), (1,)), ((), ())),
                          preferred_element_type=jnp.float32,
                          precision=lax.Precision.HIGHEST)
    acc = acc + b_ref[...]
    o_ref[...] = jnp.maximum(acc, 0.0) if relu else acc


_row_spec = pl.BlockSpec((BN, D), lambda i: (i, 0))
_p0_spec = pl.BlockSpec((1, BN, D), lambda i: (0, i, 0))
_p1_spec = pl.BlockSpec((1, BN, D), lambda i: (1, i, 0))
_full_spec = pl.BlockSpec((D, D), lambda i: (0, 0))
_b_spec = pl.BlockSpec((1, D), lambda i: (0, 0))


def _linear(x, w, b, relu=False):
    return pl.pallas_call(
        functools.partial(_mm_body, relu=relu),
        grid=(N // BN,),
        in_specs=[_row_spec, _full_spec, _b_spec],
        out_specs=_row_spec,
        out_shape=jax.ShapeDtypeStruct((N, D), jnp.float32),
    )(x, w, b.reshape(1, D))


def _linear_agg(x, p, w, b, relu=False):
    return pl.pallas_call(
        functools.partial(_mm_agg_body, relu=relu),
        grid=(N // BN,),
        in_specs=[_row_spec, _p0_spec, _p1_spec, _full_spec, _b_spec],
        out_specs=_row_spec,
        out_shape=jax.ShapeDtypeStruct((N, D), jnp.float32),
    )(x, p, p, w, b.reshape(1, D))


def kernel(x, edge_index, W_pre, b_pre, W1, b1, W2, b2):
    packed = ((edge_index[0] << 14) | edge_index[1]).reshape(NW, CH, K)
    h0 = _linear(x, W_pre, b_pre)
    p = _scatter_partials(h0, packed)
    h1 = _linear_agg(h0, p, W1, b1, relu=True)
    q = _scatter_partials(h1, packed)
    return _linear_agg(h1, q, W2, b2, relu=False)


# speedup vs baseline: 1.0893x; 1.0024x over previous
import functools

import jax
import jax.numpy as jnp
from jax import lax
from jax.experimental import pallas as pl
from jax.experimental.pallas import tpu as pltpu
from jax.experimental.pallas import tpu_sc as plsc

N = 10000
E = 320000
D = 128

NC = 2
NS = 16
NW = NC * NS
EPW = E // NW
K = 80
CH = EPW // K
RPT = 624
TAIL = N - NS * RPT
ZR = 24


def _scatter_body(h_hbm, pk_hbm, out_hbm,
                  pk, sb0, db0, sb1, db1, rows0, rows1, zbuf, acc,
                  sem0, sem1, semP, semZ):
    c = lax.axis_index("c")
    s = lax.axis_index("s")
    wid = c * NS + s

    pltpu.async_copy(pk_hbm.at[wid], pk, semP)

    for r in range(ZR):
        for q in range(D // 16):
            zbuf[r, pl.ds(q * 16, 16)] = jnp.zeros((16,), jnp.float32)
    row0 = s * RPT
    for t in range(RPT // ZR):
        pltpu.async_copy(zbuf, acc.at[pl.ds(row0 + t * ZR, ZR)], semZ)

    @pl.when(s == 0)
    def _zero_tail():
        pltpu.async_copy(zbuf.at[pl.ds(0, TAIL)], acc.at[pl.ds(NS * RPT, TAIL)],
                         semZ)

    pltpu.make_async_copy(pk_hbm.at[wid], pk, semP).wait()

    def unpack(k, sb, db):
        for q in range(K // 16):
            v = pk[k, pl.ds(q * 16, 16)]
            sb[pl.ds(q * 16, 16)] = v >> 14
            db[pl.ds(q * 16, 16)] = v & 16383

    def gather(sb, buf, sem):
        pltpu.async_copy(h_hbm.at[sb], buf, sem)

    def gwait(sb, buf, sem):
        pltpu.make_async_copy(h_hbm.at[sb], buf, sem).wait()

    def scat(buf, db):
        pltpu.sync_copy(buf, acc.at[db], add=True)

    unpack(0, sb0, db0)
    gather(sb0, rows0, sem0)

    for t in range(RPT // ZR):
        pltpu.make_async_copy(zbuf, acc.at[pl.ds(row0 + t * ZR, ZR)],
                              semZ).wait()

    @pl.when(s == 0)
    def _zero_tail_wait():
        pltpu.make_async_copy(zbuf.at[pl.ds(0, TAIL)],
                              acc.at[pl.ds(NS * RPT, TAIL)], semZ).wait()

    plsc.subcore_barrier()

    def pipe(j, carry):
        k0 = 2 * j
        unpack(k0 + 1, sb1, db1)
        gather(sb1, rows1, sem1)
        gwait(sb0, rows0, sem0)
        scat(rows0, db0)
        unpack(k0 + 2, sb0, db0)
        gather(sb0, rows0, sem0)
        gwait(sb1, rows1, sem1)
        scat(rows1, db1)
        return carry

    lax.fori_loop(0, (CH - 1) // 2, pipe, 0)
    gwait(sb0, rows0, sem0)
    scat(rows0, db0)
    plsc.subcore_barrier()

    pltpu.sync_copy(acc.at[pl.ds(row0, RPT)],
                    out_hbm.at[c, pl.ds(row0, RPT)])

    @pl.when(s == 0)
    def _write_tail():
        pltpu.sync_copy(acc.at[pl.ds(NS * RPT, TAIL)],
                        out_hbm.at[c, pl.ds(NS * RPT, TAIL)])


@jax.jit
def _scatter_partials(h, packed):
    mesh = plsc.VectorSubcoreMesh(core_axis_name="c", subcore_axis_name="s")
    f = pl.kernel(
        _scatter_body,
        out_type=jax.ShapeDtypeStruct((NC, N, D), jnp.float32),
        mesh=mesh,
        scratch_types=[
            pltpu.VMEM((CH, K), jnp.int32),
            pltpu.VMEM((K,), jnp.int32),
            pltpu.VMEM((K,), jnp.int32),
            pltpu.VMEM((K,), jnp.int32),
            pltpu.VMEM((K,), jnp.int32),
            pltpu.VMEM((K, D), jnp.float32),
            pltpu.VMEM((K, D), jnp.float32),
            pltpu.VMEM((ZR, D), jnp.float32),
            pltpu.VMEM_SHARED((N, D), jnp.float32),
            pltpu.SemaphoreType.DMA,
            pltpu.SemaphoreType.DMA,
            pltpu.SemaphoreType.DMA,
            pltpu.SemaphoreType.DMA,
        ],
    )
    return f(h, packed)


BN = 2000


def _mm_pre_body(x_ref, w_ref, b_ref, e_ref, o_ref, pk_ref):
    acc = lax.dot_general(x_ref[...], w_ref[...],
                          dimension_numbers=(((1,), (1,)), ((), ())),
                          preferred_element_type=jnp.float32,
                          precision=lax.Precision.HIGHEST)
    o_ref[...] = acc + b_ref[...]
    pk_ref[...] = (e_ref[0:1, :] << 14) | e_ref[1:2, :]


def _mm_agg_body(x_ref, p0_ref, p1_ref, w_ref, b_ref, o_ref, *, relu):
    hh = x_ref[...] + p0_ref[0] + p1_ref[0]
    acc = lax.dot_general(hh, w_ref[...],
                          dimension_numbers=(((1,), (1,)), ((), ())),
                          preferred_element_type=jnp.float32,
                          precision=lax.Precision.HIGHEST)
    acc = acc + b_ref[...]
    o_ref[...] = jnp.maximum(acc, 0.0) if relu else acc


_row_spec = pl.BlockSpec((BN, D), lambda i: (i, 0))
_p0_spec = pl.BlockSpec((1, BN, D), lambda i: (0, i, 0))
_p1_spec = pl.BlockSpec((1, BN, D), lambda i: (1, i, 0))
_full_spec = pl.BlockSpec((D, D), lambda i: (0, 0))
_b_spec = pl.BlockSpec((1, D), lambda i: (0, 0))


EB = E // (N // BN)


def _linear_pre(x, w, b, edge_index):
    return pl.pallas_call(
        _mm_pre_body,
        grid=(N // BN,),
        in_specs=[_row_spec, _full_spec, _b_spec,
                  pl.BlockSpec((2, EB), lambda i: (0, i))],
        out_specs=[_row_spec, pl.BlockSpec((1, EB), lambda i: (0, i))],
        out_shape=[jax.ShapeDtypeStruct((N, D), jnp.float32),
                   jax.ShapeDtypeStruct((1, E), jnp.int32)],
    )(x, w, b.reshape(1, D), edge_index)


def _linear_agg(x, p, w, b, relu=False):
    return pl.pallas_call(
        functools.partial(_mm_agg_body, relu=relu),
        grid=(N // BN,),
        in_specs=[_row_spec, _p0_spec, _p1_spec, _full_spec, _b_spec],
        out_specs=_row_spec,
        out_shape=jax.ShapeDtypeStruct((N, D), jnp.float32),
    )(x, p, p, w, b.reshape(1, D))


def kernel(x, edge_index, W_pre, b_pre, W1, b1, W2, b2):
    h0, packed = _linear_pre(x, W_pre, b_pre, edge_index)
    packed = packed.reshape(NW, CH, K)
    p = _scatter_partials(h0, packed)
    h1 = _linear_agg(h0, p, W1, b1, relu=True)
    q = _scatter_partials(h1, packed)
    return _linear_agg(h1, q, W2, b2, relu=False)
